# baseline (speedup 1.0000x reference)
import functools

import jax
import jax.numpy as jnp
from jax import lax
from jax.experimental import pallas as pl
from jax.experimental.pallas import tpu as pltpu
from jax.experimental.pallas import tpu_sc as plsc

N_USERS = 1000000
N_ITEMS = 1000000
EMB = 32
BATCH = 16384

_INFO = plsc.get_sparse_core_info()
_NC = _INFO.num_cores
_NS = _INFO.num_subcores
_L = _INFO.num_lanes
_NW = _NC * _NS
_BPW = BATCH // _NW
_IDX_CHUNK = 128
_NXFER = _BPW * EMB // _IDX_CHUNK

_LANES = 128
_TPC = N_USERS // _LANES
_TAIL = N_USERS - _TPC * _LANES
_STRIDE = _TPC * _LANES + _LANES
_G = 128
_NSTEP = 2
_NBAND = EMB // 8
_NBUF = 4
_RDEPTH = 2


def _sc_detile_kernel(uT3, iT3, uflat, iflat, *scratch):
    bufs = scratch[:_NBUF]
    buf_t = scratch[_NBUF]
    rsems = scratch[_NBUF + 1:2 * _NBUF + 1]
    wsems = scratch[2 * _NBUF + 1:]
    wid = lax.axis_index("s") * _NC + lax.axis_index("c")
    lo = (wid * (_TPC - _G * _NSTEP)) // (_NW - 1)

    @pl.when(wid == 0)
    def _tail():
        for src3, dstf in ((uT3, uflat), (iT3, iflat)):
            for c in range(_NBAND):
                for d in range(8):
                    pltpu.async_copy(
                        src3.at[c, d, pl.ds(_TPC * _LANES, _TAIL)],
                        buf_t, rsems[0]).wait()
                    pltpu.async_copy(
                        buf_t,
                        dstf.at[pl.ds((c * 8 + d) * _STRIDE + _TPC * _LANES,
                                      _TAIL)],
                        wsems[0]).wait()

    steps = []
    for src3, dstf in ((uT3, uflat), (iT3, iflat)):
        for c in range(_NBAND):
            for k in range(_NSTEP):
                for d in range(8):
                    steps.append((src3, dstf, c, k, d))
    nsteps = len(steps)
    pending_r = [None] * _NBUF
    pending_w = [None] * _NBUF
    for t in range(nsteps + _RDEPTH):
        if t < nsteps:
            b = t % _NBUF
            src3, dstf, c, k, d = steps[t]
            if pending_w[b] is not None:
                pending_w[b].wait()
            col0 = (lo + k * _G) * _LANES
            pending_r[b] = pltpu.async_copy(
                src3.at[c, d, pl.ds(col0, _G * _LANES)], bufs[b], rsems[b])
        tw = t - _RDEPTH
        if tw >= 0:
            bw = tw % _NBUF
            src3, dstf, c, k, d = steps[tw]
            pending_r[bw].wait()
            col0 = (lo + k * _G) * _LANES
            pending_w[bw] = pltpu.async_copy(
                bufs[bw],
                dstf.at[pl.ds((c * 8 + d) * _STRIDE + col0, _G * _LANES)],
                wsems[bw])
    for cp in pending_w:
        if cp is not None:
            cp.wait()


def _mf_kernel(u_id_hbm, i_id_hbm, uf_hbm, if_hbm, out_hbm,
               uid_v, iid_v, urows_v, irows_v, out_v, sem):
    wid = lax.axis_index("s") * _NC + lax.axis_index("c")
    base = wid * _BPW

    pltpu.sync_copy(u_id_hbm.at[pl.ds(base, _BPW)], uid_v)
    pltpu.sync_copy(i_id_hbm.at[pl.ds(base, _BPW)], iid_v)

    copies = []
    for e in range(EMB):
        dim_u = uf_hbm.at[pl.ds(e * _STRIDE, _STRIDE)]
        dim_i = if_hbm.at[pl.ds(e * _STRIDE, _STRIDE)]
        for c in range(_BPW // _IDX_CHUNK):
            sl = pl.ds(c * _IDX_CHUNK, _IDX_CHUNK)
            dsl = pl.ds(e * _BPW + c * _IDX_CHUNK, _IDX_CHUNK)
            copies.append(pltpu.async_copy(dim_u.at[uid_v.at[sl]],
                                           urows_v.at[dsl], sem))
            copies.append(pltpu.async_copy(dim_i.at[iid_v.at[sl]],
                                           irows_v.at[dsl], sem))
    for cp in copies:
        cp.wait()

    def body(g, carry):
        acc = jnp.zeros((_L,), jnp.float32)
        for e in range(EMB):
            sl = pl.ds(e * _BPW + g * _L, _L)
            acc = acc + urows_v[sl] * irows_v[sl]
        out_v[pl.ds(g * _L, _L)] = acc
        return carry

    lax.fori_loop(0, _BPW // _L, body, 0, unroll=False)

    pltpu.sync_copy(out_v, out_hbm.at[pl.ds(base, _BPW)])


@functools.partial(jax.jit)
def kernel(u_id, i_id, user_factors, item_factors):
    u_id = u_id.astype(jnp.int32)
    i_id = i_id.astype(jnp.int32)
    mesh = plsc.VectorSubcoreMesh(core_axis_name="c", subcore_axis_name="s")

    uT3 = user_factors.T.reshape(_NBAND, 8, N_USERS)
    iT3 = item_factors.T.reshape(_NBAND, 8, N_ITEMS)

    sc_detile = pl.kernel(
        _sc_detile_kernel,
        mesh=mesh,
        out_type=(jax.ShapeDtypeStruct((EMB * _STRIDE,), jnp.float32),
                  jax.ShapeDtypeStruct((EMB * _STRIDE,), jnp.float32)),
        scratch_types=(
            [pltpu.VMEM((_G * _LANES,), jnp.float32)] * _NBUF
            + [pltpu.VMEM((_TAIL,), jnp.float32)]
            + [pltpu.SemaphoreType.DMA] * (2 * _NBUF)
        ),
        compiler_params=pltpu.CompilerParams(needs_layout_passes=False),
    )
    uf_flat, if_flat = sc_detile(uT3, iT3)

    run = pl.kernel(
        _mf_kernel,
        mesh=mesh,
        out_type=jax.ShapeDtypeStruct((BATCH,), jnp.float32),
        scratch_types=[
            pltpu.VMEM((_BPW,), jnp.int32),
            pltpu.VMEM((_BPW,), jnp.int32),
            pltpu.VMEM((_BPW * EMB,), jnp.float32),
            pltpu.VMEM((_BPW * EMB,), jnp.float32),
            pltpu.VMEM((_BPW,), jnp.float32),
            pltpu.SemaphoreType.DMA,
        ],
        compiler_params=pltpu.CompilerParams(
            needs_layout_passes=False, use_tc_tiling_on_sc=False),
    )
    return run(u_id, i_id, uf_flat, if_flat)


# speedup vs baseline: 1.0183x; 1.0183x over previous
import functools

import jax
import jax.numpy as jnp
from jax import lax
from jax.experimental import pallas as pl
from jax.experimental.pallas import tpu as pltpu
from jax.experimental.pallas import tpu_sc as plsc

N_USERS = 1000000
N_ITEMS = 1000000
EMB = 32
BATCH = 16384

_INFO = plsc.get_sparse_core_info()
_NC = _INFO.num_cores
_NS = _INFO.num_subcores
_L = _INFO.num_lanes
_NW = _NC * _NS
_BPW = BATCH // _NW
_IDX_CHUNK = 128
_NXFER = _BPW * EMB // _IDX_CHUNK

_LANES = 128
_TPC = N_USERS // _LANES
_TAIL = N_USERS - _TPC * _LANES
_STRIDE = _TPC * _LANES + _LANES
_G = 64
_NSTEP = 4
_NBAND = EMB // 8
_NBUF = 8
_RDEPTH = 4


def _sc_detile_kernel(uT3, iT3, uflat, iflat, *scratch):
    bufs = scratch[:_NBUF]
    buf_t = scratch[_NBUF]
    rsems = scratch[_NBUF + 1:2 * _NBUF + 1]
    wsems = scratch[2 * _NBUF + 1:]
    wid = lax.axis_index("s") * _NC + lax.axis_index("c")
    lo = (wid * (_TPC - _G * _NSTEP)) // (_NW - 1)

    @pl.when(wid == 0)
    def _tail():
        for src3, dstf in ((uT3, uflat), (iT3, iflat)):
            for c in range(_NBAND):
                for d in range(8):
                    pltpu.async_copy(
                        src3.at[c, d, pl.ds(_TPC * _LANES, _TAIL)],
                        buf_t, rsems[0]).wait()
                    pltpu.async_copy(
                        buf_t,
                        dstf.at[pl.ds((c * 8 + d) * _STRIDE + _TPC * _LANES,
                                      _TAIL)],
                        wsems[0]).wait()

    steps = []
    for src3, dstf in ((uT3, uflat), (iT3, iflat)):
        for c in range(_NBAND):
            for k in range(_NSTEP):
                for d in range(8):
                    steps.append((src3, dstf, c, k, d))
    nsteps = len(steps)
    pending_r = [None] * _NBUF
    pending_w = [None] * _NBUF
    for t in range(nsteps + _RDEPTH):
        if t < nsteps:
            b = t % _NBUF
            src3, dstf, c, k, d = steps[t]
            if pending_w[b] is not None:
                pending_w[b].wait()
            col0 = (lo + k * _G) * _LANES
            pending_r[b] = pltpu.async_copy(
                src3.at[c, d, pl.ds(col0, _G * _LANES)], bufs[b], rsems[b])
        tw = t - _RDEPTH
        if tw >= 0:
            bw = tw % _NBUF
            src3, dstf, c, k, d = steps[tw]
            pending_r[bw].wait()
            col0 = (lo + k * _G) * _LANES
            pending_w[bw] = pltpu.async_copy(
                bufs[bw],
                dstf.at[pl.ds((c * 8 + d) * _STRIDE + col0, _G * _LANES)],
                wsems[bw])
    for cp in pending_w:
        if cp is not None:
            cp.wait()


def _mf_kernel(u_id_hbm, i_id_hbm, uf_hbm, if_hbm, out_hbm,
               uid_v, iid_v, urows_v, irows_v, out_v, sem):
    wid = lax.axis_index("s") * _NC + lax.axis_index("c")
    base = wid * _BPW

    pltpu.sync_copy(u_id_hbm.at[pl.ds(base, _BPW)], uid_v)
    pltpu.sync_copy(i_id_hbm.at[pl.ds(base, _BPW)], iid_v)

    copies = []
    for e in range(EMB):
        dim_u = uf_hbm.at[pl.ds(e * _STRIDE, _STRIDE)]
        dim_i = if_hbm.at[pl.ds(e * _STRIDE, _STRIDE)]
        for c in range(_BPW // _IDX_CHUNK):
            sl = pl.ds(c * _IDX_CHUNK, _IDX_CHUNK)
            dsl = pl.ds(e * _BPW + c * _IDX_CHUNK, _IDX_CHUNK)
            copies.append(pltpu.async_copy(dim_u.at[uid_v.at[sl]],
                                           urows_v.at[dsl], sem))
            copies.append(pltpu.async_copy(dim_i.at[iid_v.at[sl]],
                                           irows_v.at[dsl], sem))
    for cp in copies:
        cp.wait()

    def body(g, carry):
        acc = jnp.zeros((_L,), jnp.float32)
        for e in range(EMB):
            sl = pl.ds(e * _BPW + g * _L, _L)
            acc = acc + urows_v[sl] * irows_v[sl]
        out_v[pl.ds(g * _L, _L)] = acc
        return carry

    lax.fori_loop(0, _BPW // _L, body, 0, unroll=False)

    pltpu.sync_copy(out_v, out_hbm.at[pl.ds(base, _BPW)])


@functools.partial(jax.jit)
def kernel(u_id, i_id, user_factors, item_factors):
    u_id = u_id.astype(jnp.int32)
    i_id = i_id.astype(jnp.int32)
    mesh = plsc.VectorSubcoreMesh(core_axis_name="c", subcore_axis_name="s")

    uT3 = user_factors.T.reshape(_NBAND, 8, N_USERS)
    iT3 = item_factors.T.reshape(_NBAND, 8, N_ITEMS)

    sc_detile = pl.kernel(
        _sc_detile_kernel,
        mesh=mesh,
        out_type=(jax.ShapeDtypeStruct((EMB * _STRIDE,), jnp.float32),
                  jax.ShapeDtypeStruct((EMB * _STRIDE,), jnp.float32)),
        scratch_types=(
            [pltpu.VMEM((_G * _LANES,), jnp.float32)] * _NBUF
            + [pltpu.VMEM((_TAIL,), jnp.float32)]
            + [pltpu.SemaphoreType.DMA] * (2 * _NBUF)
        ),
        compiler_params=pltpu.CompilerParams(needs_layout_passes=False),
    )
    uf_flat, if_flat = sc_detile(uT3, iT3)

    run = pl.kernel(
        _mf_kernel,
        mesh=mesh,
        out_type=jax.ShapeDtypeStruct((BATCH,), jnp.float32),
        scratch_types=[
            pltpu.VMEM((_BPW,), jnp.int32),
            pltpu.VMEM((_BPW,), jnp.int32),
            pltpu.VMEM((_BPW * EMB,), jnp.float32),
            pltpu.VMEM((_BPW * EMB,), jnp.float32),
            pltpu.VMEM((_BPW,), jnp.float32),
            pltpu.SemaphoreType.DMA,
        ],
        compiler_params=pltpu.CompilerParams(
            needs_layout_passes=False, use_tc_tiling_on_sc=False),
    )
    return run(u_id, i_id, uf_flat, if_flat)
